# baseline (speedup 1.0000x reference)
import functools
import math

import jax
import jax.numpy as jnp
from jax import lax
from jax.experimental import pallas as pl
from jax.experimental.pallas import tpu as pltpu
from jax.experimental.pallas import tpu_sc as plsc

SEQ = 256
LEN = 256
QH = 128
DM = 256
DP = 288
NV = 22
VP = 32
MAXG = 32
NIDX = 65
NWORK = 32
JW = LEN // NWORK
ICH = 16


def _prep_body(encT_ref, seq_ref, res_ref, qtab_ref, mtab_ref, ltab_ref,
               rtab_ref, ppW_ref, ppb_ref, qrnb_ref, qrlW_ref, qrlb_ref,
               prnb_ref, prlW_ref, prlb_ref,
               qpart_ref, cosv_ref, q0_ref, a_ref, bp_ref, epad_ref):
    f32 = jnp.float32
    i32 = jnp.int32
    iota_v = lax.broadcasted_iota(i32, (LEN, VP), 1)
    oh_seq = (seq_ref[:] == iota_v).astype(f32)
    q_emb = jnp.dot(oh_seq, qtab_ref[:], preferred_element_type=f32)
    lf = res_ref[:].astype(f32)
    kk = lax.broadcasted_iota(i32, (1, QH), 1).astype(f32)
    inv_freq = jnp.exp(kk * (-math.log(10000.0) / QH))
    ang = lf * inv_freq
    sinv = jnp.sin(ang)
    cosv = jnp.cos(ang)
    qpart = q_emb + sinv
    qpart_ref[:] = qpart
    cosv_ref[:] = cosv
    e0 = encT_ref[:, 0:1]
    oh0 = (e0 == iota_v).astype(f32)
    m0 = jnp.dot(oh0, mtab_ref[:], preferred_element_type=f32) + cosv
    row0 = jnp.concatenate([qpart, m0], axis=1)
    q0_ref[:] = (jnp.dot(row0, qrlW_ref[0:DM, :], preferred_element_type=f32)
                 + jnp.dot(qrnb_ref[:], qrlW_ref[DM:2 * DM, :],
                           preferred_element_type=f32)
                 + qrlb_ref[:])
    w1 = prlW_ref[0:DP, :]
    lw = jnp.dot(ltab_ref[:], w1, preferred_element_type=f32)
    rw = jnp.dot(rtab_ref[:], w1, preferred_element_type=f32)
    a_ref[:] = jnp.dot(oh_seq, lw, preferred_element_type=f32)
    constv = (jnp.dot(ppb_ref[:], w1, preferred_element_type=f32)
              + jnp.dot(prnb_ref[:], prlW_ref[DP:2 * DP, :],
                        preferred_element_type=f32)
              + prlb_ref[:])
    bp_ref[:] = jnp.dot(oh_seq, rw, preferred_element_type=f32) + constv
    cmat = jnp.dot(ppW_ref[:], w1, preferred_element_type=f32)
    mm = lax.broadcasted_iota(i32, (2 * LEN, NIDX), 0)
    cc = lax.broadcasted_iota(i32, (2 * LEN, NIDX), 1)
    dd = jnp.clip(mm - (LEN - 1), -MAXG, MAXG) + MAXG
    oh_e = (dd == cc).astype(f32)
    epad_ref[:] = jnp.dot(oh_e, cmat, preferred_element_type=f32)


def _msa_body(enc_ref, tabT_ref, qpart_ref, cosv_ref, q0_ref, o_ref):
    f32 = jnp.float32
    i32 = jnp.int32
    i = pl.program_id(0)
    enc = enc_ref[0]
    iota_v = lax.broadcasted_iota(i32, (VP, 1), 0)
    ohT = (enc == iota_v).astype(f32)
    halfT = jnp.dot(tabT_ref[:], ohT, preferred_element_type=f32)
    half = halfT.T.reshape(16, LEN, QH)
    o_ref[0, :, :, 0:QH] = jnp.broadcast_to(qpart_ref[:][None], (16, LEN, QH))
    o_ref[0, :, :, QH:DM] = half + cosv_ref[:][None]

    @pl.when(i == 0)
    def _():
        o_ref[0, 0] = q0_ref[:]


def _sc_pair_body(a_hbm, bp_hbm, e_hbm, out_hbm, bp_v, a_v, e_v, o_v):
    wid = lax.axis_index("s") * 2 + lax.axis_index("c")
    j0 = wid * JW
    pltpu.sync_copy(bp_hbm.at[pl.ds(j0, JW)], bp_v)

    def chunk(ci, _):
        ibase = ci * ICH
        pltpu.sync_copy(a_hbm.at[pl.ds(ibase, ICH)], a_v)
        s0 = (LEN - ICH) - ibase + j0
        pltpu.sync_copy(e_hbm.at[pl.ds(s0, ICH + JW)], e_v)

        def row(ii, _):
            er0 = (ICH - 1) - ii
            for jj in range(JW):
                for k in range(DP // 16):
                    sl = pl.ds(16 * k, 16)
                    o_v[ii, jj, sl] = (a_v[ii, sl] + bp_v[jj, sl]) + e_v[er0 + jj, sl]
            return 0

        lax.fori_loop(0, ICH, row, 0)
        pltpu.sync_copy(o_v, out_hbm.at[0, pl.ds(ibase, ICH), pl.ds(j0, JW)])
        return 0

    lax.fori_loop(0, LEN // ICH, chunk, 0)


def _pad_rows(t):
    return jnp.concatenate(
        [t, jnp.zeros((VP - t.shape[0], t.shape[1]), t.dtype)], axis=0)


def kernel(MSA_encoding, seq_encoding, res_idxs, MSA_table, query_table,
           left_table, right_table, pos_pair_W, pos_pair_b,
           qrn_g, qrn_b, qrl_W, qrl_b, prn_g, prn_b, prl_W, prl_b):
    enc = MSA_encoding[0].astype(jnp.int32)
    encT = enc.T
    seq2 = seq_encoding[0].astype(jnp.int32).reshape(LEN, 1)
    res2 = res_idxs[0].astype(jnp.int32).reshape(LEN, 1)

    qpart, cosv, q0, amat, bpmat, epad = pl.pallas_call(
        _prep_body,
        out_shape=[
            jax.ShapeDtypeStruct((LEN, QH), jnp.float32),
            jax.ShapeDtypeStruct((LEN, QH), jnp.float32),
            jax.ShapeDtypeStruct((LEN, DM), jnp.float32),
            jax.ShapeDtypeStruct((LEN, DP), jnp.float32),
            jax.ShapeDtypeStruct((LEN, DP), jnp.float32),
            jax.ShapeDtypeStruct((2 * LEN, DP), jnp.float32),
        ],
    )(encT, seq2, res2,
      _pad_rows(query_table), _pad_rows(MSA_table),
      _pad_rows(left_table), _pad_rows(right_table),
      pos_pair_W, pos_pair_b.reshape(1, DP),
      qrn_b.reshape(1, DM), qrl_W, qrl_b.reshape(1, DM),
      prn_b.reshape(1, DP), prl_W, prl_b.reshape(1, DP))

    msa = pl.pallas_call(
        _msa_body,
        grid=(SEQ // 16,),
        in_specs=[
            pl.BlockSpec((1, 1, 16 * LEN), lambda i: (i, 0, 0)),
            pl.BlockSpec((QH, VP), lambda i: (0, 0)),
            pl.BlockSpec((LEN, QH), lambda i: (0, 0)),
            pl.BlockSpec((LEN, QH), lambda i: (0, 0)),
            pl.BlockSpec((LEN, DM), lambda i: (0, 0)),
        ],
        out_specs=pl.BlockSpec((1, 16, LEN, DM), lambda i: (0, i, 0, 0)),
        out_shape=jax.ShapeDtypeStruct((1, SEQ, LEN, DM), jnp.float32),
    )(enc.reshape(SEQ // 16, 1, 16 * LEN), _pad_rows(MSA_table).T,
      qpart, cosv, q0)

    sc_pair = pl.kernel(
        _sc_pair_body,
        mesh=plsc.VectorSubcoreMesh(
            core_axis_name="c", subcore_axis_name="s", num_cores=2),
        out_type=jax.ShapeDtypeStruct((1, LEN, LEN, DP), jnp.float32),
        scratch_types=[
            pltpu.VMEM((JW, DP), jnp.float32),
            pltpu.VMEM((ICH, DP), jnp.float32),
            pltpu.VMEM((ICH + JW, DP), jnp.float32),
            pltpu.VMEM((ICH, JW, DP), jnp.float32),
        ],
    )
    pair = sc_pair(amat, bpmat, epad)

    return (msa, pair)
